# baseline (speedup 1.0000x reference)
import jax
import jax.numpy as jnp
import numpy as np
from jax.experimental import pallas as pl

N = 4 * 512 * 512
R, C = 8192, 128
GRID = 16
BR = R // GRID
PAD = np.float32(2.0)
ONE_SIGMA = np.float32(1.15)


def _part(gv, gi, pv, pi, base_mask):
    flag1 = gv / gi
    flag2 = gi / gv
    target = jnp.where(flag1 >= ONE_SIGMA, jnp.float32(1.0), jnp.float32(0.0))
    target = jnp.where(flag2 > ONE_SIGMA, jnp.float32(-1.0), target)
    diff = pv - pi
    nz = jnp.logical_and(target != 0.0, base_mask)
    z = jnp.logical_and(target == 0.0, base_mask)
    log_terms = jnp.log(1.0 + jnp.exp(-target * diff))
    s_log = jnp.sum(jnp.where(nz, log_terms, 0.0))
    c_nz = jnp.sum(nz.astype(jnp.float32))
    s_sq = jnp.sum(jnp.where(z, diff * diff, 0.0))
    c_z = jnp.sum(z.astype(jnp.float32))
    return s_log, c_nz, s_sq, c_z


def _rowshift2(x):
    return jnp.concatenate([x[:, 2:], x[:, :2]], axis=1)


def _loss_kernel(s_ref, gn_ref, gfn_ref, pn_ref, pfn_ref, out_ref):
    step = pl.program_id(0)
    half = s_ref[0, 0]
    ms = s_ref[0, 1]

    gn = gn_ref[...]
    gf = -gfn_ref[...]
    pn = pn_ref[...]
    pf = -pfn_ref[...]

    row = jax.lax.broadcasted_iota(jnp.int32, (BR, C), 0) + step * BR
    lane = jax.lax.broadcasted_iota(jnp.int32, (BR, C), 1)
    p = row * C + lane
    mask12 = jnp.logical_and((p % 4) == 0, (p // 4) < half)
    mask3 = jnp.logical_and((p % 2) == 1, ((p - 1) // 2) < ms)

    r1 = _part(gn, _rowshift2(gn), pn, _rowshift2(pn), mask12)
    r2 = _part(gf, _rowshift2(gf), pf, _rowshift2(pf), mask12)
    r3 = _part(gn, gf, pn, pf, mask3)

    sums = [r1[k] + r2[k] + r3[k] for k in range(4)]
    slot = jax.lax.broadcasted_iota(jnp.int32, (1, 4), 1)
    vec = sum(sums[k] * (slot == k).astype(jnp.float32) for k in range(4))

    @pl.when(step == 0)
    def _init():
        out_ref[...] = vec

    @pl.when(step != 0)
    def _acc():
        out_ref[...] = out_ref[...] + vec


def kernel(pred_depth, gt_depth):
    pred = pred_depth.reshape(-1)
    depth = gt_depth.reshape(-1)
    valid = depth > 0

    keys_d = jnp.stack([jnp.where(valid, depth, PAD),
                        jnp.where(valid, -depth, PAD)])
    sd = jnp.sort(keys_d, axis=1)
    gn = sd[0]
    gfn = sd[1]

    n_pos = jnp.sum(valid)
    q_index = jnp.float32(0.75) * (n_pos - 1)
    n_pos_f = (n_pos - 1).astype(jnp.float32)
    low = jnp.clip(jnp.floor(q_index), 0, n_pos_f)
    high = jnp.clip(jnp.ceil(q_index), 0, n_pos_f)
    high_weight = q_index - low
    low_weight = jnp.float32(1) - high_weight
    low_value = jnp.take(gn, low.astype(jnp.int32))
    high_value = jnp.take(gn, high.astype(jnp.int32))
    thre = low_value * low_weight + high_value * high_weight

    mask_A = jnp.logical_and(depth <= thre, valid)
    mask_B = depth > thre
    keys_p = jnp.stack([jnp.where(mask_A, pred, PAD),
                        jnp.where(mask_B, -pred, PAD)])
    sp = jnp.sort(keys_p, axis=1)
    pn = sp[0]
    pfn = sp[1]

    n_a = jnp.sum(mask_A)
    n_b = jnp.sum(mask_B)
    m = jnp.minimum(n_a, n_b)
    scalars = jnp.stack([m // 4, m // 2]).reshape(1, 2).astype(jnp.int32)

    sums = pl.pallas_call(
        _loss_kernel,
        grid=(GRID,),
        in_specs=[
            pl.BlockSpec((1, 2), lambda i: (0, 0)),
            pl.BlockSpec((BR, C), lambda i: (i, 0)),
            pl.BlockSpec((BR, C), lambda i: (i, 0)),
            pl.BlockSpec((BR, C), lambda i: (i, 0)),
            pl.BlockSpec((BR, C), lambda i: (i, 0)),
        ],
        out_specs=pl.BlockSpec((1, 4), lambda i: (0, 0)),
        out_shape=jax.ShapeDtypeStruct((1, 4), jnp.float32),
    )(scalars, gn.reshape(R, C), gfn.reshape(R, C),
      pn.reshape(R, C), pfn.reshape(R, C))

    log_loss = sums[0, 0] / sums[0, 1]
    squared_loss = sums[0, 2] / sums[0, 3]
    loss = jnp.where(jnp.isnan(log_loss), squared_loss,
                     jnp.where(jnp.isnan(squared_loss), log_loss,
                               log_loss + squared_loss))
    return jnp.reshape(loss, (1,)).astype(jnp.float32)


# speedup vs baseline: 6.7828x; 6.7828x over previous
import jax
import jax.numpy as jnp
import numpy as np
from jax.experimental import pallas as pl

N = 4 * 512 * 512
R, C = 8192, 128
GRID = 16
BR = R // GRID
PAD = np.float32(2.0)
ONE_SIGMA = np.float32(1.15)


def _part(gv, gi, pv, pi, base_mask):
    flag1 = gv / gi
    flag2 = gi / gv
    target = jnp.where(flag1 >= ONE_SIGMA, jnp.float32(1.0), jnp.float32(0.0))
    target = jnp.where(flag2 > ONE_SIGMA, jnp.float32(-1.0), target)
    diff = pv - pi
    nz = jnp.logical_and(target != 0.0, base_mask)
    z = jnp.logical_and(target == 0.0, base_mask)
    log_terms = jnp.log(1.0 + jnp.exp(-target * diff))
    s_log = jnp.sum(jnp.where(nz, log_terms, 0.0))
    c_nz = jnp.sum(nz.astype(jnp.float32))
    s_sq = jnp.sum(jnp.where(z, diff * diff, 0.0))
    c_z = jnp.sum(z.astype(jnp.float32))
    return s_log, c_nz, s_sq, c_z


def _rowshift2(x):
    return jnp.concatenate([x[:, 2:], x[:, :2]], axis=1)


def _loss_kernel(s_ref, gn_ref, gf_ref, pn_ref, pf_ref, out_ref):
    step = pl.program_id(0)
    half = s_ref[0, 0]
    ms = s_ref[0, 1]

    gn = gn_ref[...]
    gf = gf_ref[...]
    pn = pn_ref[...]
    pf = pf_ref[...]

    row = jax.lax.broadcasted_iota(jnp.int32, (BR, C), 0) + step * BR
    lane = jax.lax.broadcasted_iota(jnp.int32, (BR, C), 1)
    p = row * C + lane
    mask12 = jnp.logical_and((p % 4) == 0, (p // 4) < half)
    mask3 = jnp.logical_and((p % 2) == 1, ((p - 1) // 2) < ms)

    r1 = _part(gn, _rowshift2(gn), pn, _rowshift2(pn), mask12)
    r2 = _part(gf, _rowshift2(gf), pf, _rowshift2(pf), mask12)
    r3 = _part(gn, gf, pn, pf, mask3)

    sums = [r1[k] + r2[k] + r3[k] for k in range(4)]
    slot = jax.lax.broadcasted_iota(jnp.int32, (1, 4), 1)
    vec = sum(sums[k] * (slot == k).astype(jnp.float32) for k in range(4))

    @pl.when(step == 0)
    def _init():
        out_ref[...] = vec

    @pl.when(step != 0)
    def _acc():
        out_ref[...] = out_ref[...] + vec


def kernel(pred_depth, gt_depth):
    pred = pred_depth.reshape(-1)
    depth = gt_depth.reshape(-1)
    valid = depth > 0

    gn = jnp.sort(jnp.where(valid, depth, PAD))
    n_pos = jnp.sum(valid)
    gf = jnp.roll(gn[::-1], -(N - n_pos))
    q_index = jnp.float32(0.75) * (n_pos - 1)
    n_pos_f = (n_pos - 1).astype(jnp.float32)
    low = jnp.clip(jnp.floor(q_index), 0, n_pos_f)
    high = jnp.clip(jnp.ceil(q_index), 0, n_pos_f)
    high_weight = q_index - low
    low_weight = jnp.float32(1) - high_weight
    low_value = jnp.take(gn, low.astype(jnp.int32))
    high_value = jnp.take(gn, high.astype(jnp.int32))
    thre = low_value * low_weight + high_value * high_weight

    mask_A = jnp.logical_and(depth <= thre, valid)
    mask_B = depth > thre
    sp = jnp.sort(jnp.where(mask_B, -pred, jnp.where(mask_A, pred, PAD)))
    n_a = jnp.sum(mask_A)
    n_b = jnp.sum(mask_B)
    pf = -sp
    pn = jnp.roll(sp, -n_b)
    m = jnp.minimum(n_a, n_b)
    scalars = jnp.stack([m // 4, m // 2]).reshape(1, 2).astype(jnp.int32)

    sums = pl.pallas_call(
        _loss_kernel,
        grid=(GRID,),
        in_specs=[
            pl.BlockSpec((1, 2), lambda i: (0, 0)),
            pl.BlockSpec((BR, C), lambda i: (i, 0)),
            pl.BlockSpec((BR, C), lambda i: (i, 0)),
            pl.BlockSpec((BR, C), lambda i: (i, 0)),
            pl.BlockSpec((BR, C), lambda i: (i, 0)),
        ],
        out_specs=pl.BlockSpec((1, 4), lambda i: (0, 0)),
        out_shape=jax.ShapeDtypeStruct((1, 4), jnp.float32),
    )(scalars, gn.reshape(R, C), gf.reshape(R, C),
      pn.reshape(R, C), pf.reshape(R, C))

    log_loss = sums[0, 0] / sums[0, 1]
    squared_loss = sums[0, 2] / sums[0, 3]
    loss = jnp.where(jnp.isnan(log_loss), squared_loss,
                     jnp.where(jnp.isnan(squared_loss), log_loss,
                               log_loss + squared_loss))
    return jnp.reshape(loss, (1,)).astype(jnp.float32)
